# baseline (speedup 1.0000x reference)
import jax
import jax.numpy as jnp
from jax import lax
from jax.experimental import pallas as pl
from jax.experimental.pallas import tpu as pltpu

T = 256
SUB = 8


def _sigmoid(v):
    return 1.0 / (1.0 + jnp.exp(-v))


def _softplus(v):
    return jnp.maximum(v, 0.0) + jnp.log1p(jnp.exp(-jnp.abs(v)))


def _make_kernel(B, DM, DI, N, R, L, nch, t8):
    def body(x_ref, xh_ref, wiu_ref, wiz_ref, biu_ref, biz_ref, cw_ref,
             cb_ref, wdtr_ref, wb_ref, wc_ref, wdt_ref, dtb_ref, at_ref,
             d_ref, wo_ref, bo_ref, out_ref,
             g_s, uc_s, delta_s, bc_s, cc_s, y_s, h_s, ucar):
        j = pl.program_id(1)
        xc = x_ref[0]

        u_raw = jnp.dot(xc, wiu_ref[...],
                        preferred_element_type=jnp.float32) + biu_ref[...]
        zv = jnp.dot(xc, wiz_ref[...],
                     preferred_element_type=jnp.float32) + biz_ref[...]
        g_s[...] = zv * _sigmoid(zv)

        prev = jnp.where(j == 0, 0.0, ucar[...])
        nxt = jnp.dot(xh_ref[0, 0:1, :], wiu_ref[...],
                      preferred_element_type=jnp.float32) + biu_ref[...]
        nxt = jnp.where(j == nch - 1, 0.0, nxt)
        ucar[...] = u_raw[T - 1:T, :]
        u_dn = jnp.concatenate([prev, u_raw[:T - 1, :]], axis=0)
        u_up = jnp.concatenate([u_raw[1:, :], nxt], axis=0)
        ucv = (u_dn * cw_ref[0:1, :] + u_raw * cw_ref[1:2, :]
               + u_up * cw_ref[2:3, :] + cb_ref[...])
        ucv = ucv * _sigmoid(ucv)
        uc_s[...] = ucv.reshape(t8, SUB, DI)

        dn = (((1,), (1,)), ((), ()))
        dtr = lax.dot_general(ucv, wdtr_ref[...], dn,
                              preferred_element_type=jnp.float32)
        bc = lax.dot_general(ucv, wb_ref[...], dn,
                             preferred_element_type=jnp.float32)
        cc = lax.dot_general(ucv, wc_ref[...], dn,
                             preferred_element_type=jnp.float32)
        bc_s[...] = bc.reshape(t8, SUB, N)
        cc_s[...] = cc.reshape(t8, SUB, N)
        dtpre = jnp.dot(dtr, wdt_ref[...],
                        preferred_element_type=jnp.float32) + 2.0 * dtb_ref[...]
        delta_s[...] = _softplus(dtpre).reshape(t8, SUB, DI)

        aneg = -jnp.exp(at_ref[...])

        @pl.when(j == 0)
        def _():
            h_s[...] = jnp.zeros_like(h_s)

        def block(tb, h):
            d8 = delta_s[tb]
            du8 = d8 * uc_s[tb]
            bt8 = jnp.transpose(bc_s[tb], (1, 0))
            ct8 = jnp.transpose(cc_s[tb], (1, 0))
            rows = []
            for r in range(SUB):
                da = jnp.exp(d8[r:r + 1, :] * aneg)
                dbu = du8[r:r + 1, :] * bt8[:, r:r + 1]
                h = da * h + dbu
                rows.append(jnp.sum(h * ct8[:, r:r + 1], axis=0,
                                    keepdims=True))
            y_s[tb] = jnp.concatenate(rows, axis=0)
            return h

        h = lax.fori_loop(0, t8, block, h_s[...])
        h_s[...] = h

        y = y_s[...].reshape(T, DI)
        yg = (y + d_ref[...] * uc_s[...].reshape(T, DI)) * g_s[...]
        out_ref[0] = (jnp.dot(yg, wo_ref[...],
                              preferred_element_type=jnp.float32)
                      + bo_ref[...] + xc)
    return body


def kernel(x, in_proj_w, in_proj_b, conv_w, conv_b, x_proj_w, dt_proj_w,
           dt_proj_b, A_log, D, out_proj_w, out_proj_b):
    B, DM, L = x.shape
    DI = in_proj_w.shape[0] // 2
    R = dt_proj_w.shape[1]
    N = (x_proj_w.shape[0] - R) // 2
    nch = L // T
    t8 = T // SUB

    x_t = jnp.transpose(x, (0, 2, 1))
    wiu = jnp.transpose(in_proj_w[:DI], (1, 0))
    wiz = jnp.transpose(in_proj_w[DI:], (1, 0))
    biu = in_proj_b[:DI][None, :]
    biz = in_proj_b[DI:][None, :]
    cw = jnp.transpose(conv_w[:, 0, :], (1, 0))
    cb = conv_b[None, :]
    wdtr = x_proj_w[:R]
    wb = x_proj_w[R:R + N]
    wc = x_proj_w[R + N:]
    wdt = jnp.transpose(dt_proj_w, (1, 0))
    dtb = dt_proj_b[None, :]
    at = jnp.transpose(A_log, (1, 0))
    drow = D[None, :]
    wo = jnp.transpose(out_proj_w, (1, 0))
    bo = out_proj_b[None, :]

    full = lambda s: pl.BlockSpec(s, lambda b, j: tuple(0 for _ in s))
    out_t = pl.pallas_call(
        _make_kernel(B, DM, DI, N, R, L, nch, t8),
        out_shape=jax.ShapeDtypeStruct((B, L, DM), jnp.float32),
        grid=(B, nch),
        in_specs=[
            pl.BlockSpec((1, T, DM), lambda b, j: (b, j, 0)),
            pl.BlockSpec((1, SUB, DM),
                         lambda b, j: (b, jnp.minimum((j + 1) * (T // SUB),
                                                      L // SUB - 1), 0)),
            full((DM, DI)), full((DM, DI)), full((1, DI)), full((1, DI)),
            full((3, DI)), full((1, DI)), full((R, DI)), full((N, DI)),
            full((N, DI)), full((R, DI)), full((1, DI)), full((N, DI)),
            full((1, DI)), full((DI, DM)), full((1, DM)),
        ],
        out_specs=pl.BlockSpec((1, T, DM), lambda b, j: (b, j, 0)),
        scratch_shapes=[
            pltpu.VMEM((T, DI), jnp.float32),
            pltpu.VMEM((t8, SUB, DI), jnp.float32),
            pltpu.VMEM((t8, SUB, DI), jnp.float32),
            pltpu.VMEM((t8, SUB, N), jnp.float32),
            pltpu.VMEM((t8, SUB, N), jnp.float32),
            pltpu.VMEM((t8, SUB, DI), jnp.float32),
            pltpu.VMEM((N, DI), jnp.float32),
            pltpu.VMEM((1, DI), jnp.float32),
        ],
        compiler_params=pltpu.CompilerParams(
            dimension_semantics=("parallel", "arbitrary"),
            vmem_limit_bytes=64 * 1024 * 1024,
        ),
        name="vsss_block1d",
    )(x_t, x_t, wiu, wiz, biu, biz, cw, cb, wdtr, wb, wc, wdt, dtb, at,
      drow, wo, bo)
    return jnp.transpose(out_t, (0, 2, 1))


# speedup vs baseline: 1.0676x; 1.0676x over previous
import jax
import jax.numpy as jnp
from jax import lax
from jax.experimental import pallas as pl
from jax.experimental.pallas import tpu as pltpu

T = 256
SUB = 4


def _sigmoid(v):
    return 1.0 / (1.0 + jnp.exp(-v))


def _softplus(v):
    return jnp.maximum(v, 0.0) + jnp.log1p(jnp.exp(-jnp.abs(v)))


def _make_kernel(B, DM, DI, N, R, L, nch, t8):
    def body(x_ref, xh_ref, wiu_ref, wiz_ref, biu_ref, biz_ref, cw_ref,
             cb_ref, wdtr_ref, wb_ref, wc_ref, wdt_ref, dtb_ref, at_ref,
             d_ref, wo_ref, bo_ref, out_ref,
             g_s, uc_s, delta_s, bc_s, cc_s, y_s, h_s, ucar):
        j = pl.program_id(1)
        xc = x_ref[0]

        u_raw = jnp.dot(xc, wiu_ref[...],
                        preferred_element_type=jnp.float32) + biu_ref[...]
        zv = jnp.dot(xc, wiz_ref[...],
                     preferred_element_type=jnp.float32) + biz_ref[...]
        g_s[...] = zv * _sigmoid(zv)

        prev = jnp.where(j == 0, 0.0, ucar[...])
        nxt = jnp.dot(xh_ref[0, 0:1, :], wiu_ref[...],
                      preferred_element_type=jnp.float32) + biu_ref[...]
        nxt = jnp.where(j == nch - 1, 0.0, nxt)
        ucar[...] = u_raw[T - 1:T, :]
        u_dn = jnp.concatenate([prev, u_raw[:T - 1, :]], axis=0)
        u_up = jnp.concatenate([u_raw[1:, :], nxt], axis=0)
        ucv = (u_dn * cw_ref[0:1, :] + u_raw * cw_ref[1:2, :]
               + u_up * cw_ref[2:3, :] + cb_ref[...])
        ucv = ucv * _sigmoid(ucv)
        uc_s[...] = ucv.reshape(t8, SUB, DI)

        dn = (((1,), (1,)), ((), ()))
        dtr = lax.dot_general(ucv, wdtr_ref[...], dn,
                              preferred_element_type=jnp.float32)
        bc = lax.dot_general(ucv, wb_ref[...], dn,
                             preferred_element_type=jnp.float32)
        cc = lax.dot_general(ucv, wc_ref[...], dn,
                             preferred_element_type=jnp.float32)
        bc_s[...] = bc.reshape(t8, SUB, N)
        cc_s[...] = cc.reshape(t8, SUB, N)
        dtpre = jnp.dot(dtr, wdt_ref[...],
                        preferred_element_type=jnp.float32) + 2.0 * dtb_ref[...]
        delta_s[...] = _softplus(dtpre).reshape(t8, SUB, DI)

        aneg = -jnp.exp(at_ref[...])

        @pl.when(j == 0)
        def _():
            h_s[...] = jnp.zeros_like(h_s)

        def block(tb, h):
            d8 = delta_s[tb]
            u8 = uc_s[tb]
            b8 = bc_s[tb]
            c8 = cc_s[tb]
            du8 = d8 * u8
            da8 = jnp.exp(d8[:, None, :] * aneg[None, :, :])
            db8 = du8[:, None, :] * b8[:, :, None]
            c83 = c8[:, :, None]
            rows = []
            for r in range(SUB):
                h = da8[r] * h + db8[r]
                rows.append(jnp.sum(h * c83[r], axis=0, keepdims=True))
            y_s[tb] = jnp.concatenate(rows, axis=0)
            return h

        h = lax.fori_loop(0, t8, block, h_s[...])
        h_s[...] = h

        y = y_s[...].reshape(T, DI)
        yg = (y + d_ref[...] * uc_s[...].reshape(T, DI)) * g_s[...]
        out_ref[0] = (jnp.dot(yg, wo_ref[...],
                              preferred_element_type=jnp.float32)
                      + bo_ref[...] + xc)
    return body


def kernel(x, in_proj_w, in_proj_b, conv_w, conv_b, x_proj_w, dt_proj_w,
           dt_proj_b, A_log, D, out_proj_w, out_proj_b):
    B, DM, L = x.shape
    DI = in_proj_w.shape[0] // 2
    R = dt_proj_w.shape[1]
    N = (x_proj_w.shape[0] - R) // 2
    nch = L // T
    t8 = T // SUB

    x_t = jnp.transpose(x, (0, 2, 1))
    wiu = jnp.transpose(in_proj_w[:DI], (1, 0))
    wiz = jnp.transpose(in_proj_w[DI:], (1, 0))
    biu = in_proj_b[:DI][None, :]
    biz = in_proj_b[DI:][None, :]
    cw = jnp.transpose(conv_w[:, 0, :], (1, 0))
    cb = conv_b[None, :]
    wdtr = x_proj_w[:R]
    wb = x_proj_w[R:R + N]
    wc = x_proj_w[R + N:]
    wdt = jnp.transpose(dt_proj_w, (1, 0))
    dtb = dt_proj_b[None, :]
    at = jnp.transpose(A_log, (1, 0))
    drow = D[None, :]
    wo = jnp.transpose(out_proj_w, (1, 0))
    bo = out_proj_b[None, :]

    full = lambda s: pl.BlockSpec(s, lambda b, j: tuple(0 for _ in s))
    out_t = pl.pallas_call(
        _make_kernel(B, DM, DI, N, R, L, nch, t8),
        out_shape=jax.ShapeDtypeStruct((B, L, DM), jnp.float32),
        grid=(B, nch),
        in_specs=[
            pl.BlockSpec((1, T, DM), lambda b, j: (b, j, 0)),
            pl.BlockSpec((1, 8, DM),
                         lambda b, j: (b, jnp.minimum((j + 1) * (T // 8),
                                                      L // 8 - 1), 0)),
            full((DM, DI)), full((DM, DI)), full((1, DI)), full((1, DI)),
            full((3, DI)), full((1, DI)), full((R, DI)), full((N, DI)),
            full((N, DI)), full((R, DI)), full((1, DI)), full((N, DI)),
            full((1, DI)), full((DI, DM)), full((1, DM)),
        ],
        out_specs=pl.BlockSpec((1, T, DM), lambda b, j: (b, j, 0)),
        scratch_shapes=[
            pltpu.VMEM((T, DI), jnp.float32),
            pltpu.VMEM((t8, SUB, DI), jnp.float32),
            pltpu.VMEM((t8, SUB, DI), jnp.float32),
            pltpu.VMEM((t8, SUB, N), jnp.float32),
            pltpu.VMEM((t8, SUB, N), jnp.float32),
            pltpu.VMEM((t8, SUB, DI), jnp.float32),
            pltpu.VMEM((N, DI), jnp.float32),
            pltpu.VMEM((1, DI), jnp.float32),
        ],
        compiler_params=pltpu.CompilerParams(
            dimension_semantics=("parallel", "arbitrary"),
            vmem_limit_bytes=64 * 1024 * 1024,
        ),
        name="vsss_block1d",
    )(x_t, x_t, wiu, wiz, biu, biz, cw, cb, wdtr, wb, wc, wdt, dtb, at,
      drow, wo, bo)
    return jnp.transpose(out_t, (0, 2, 1))


# speedup vs baseline: 1.1268x; 1.0554x over previous
import jax
import jax.numpy as jnp
from jax import lax
from jax.experimental import pallas as pl
from jax.experimental.pallas import tpu as pltpu

T = 256
SUB = 8


def _sigmoid(v):
    return 1.0 / (1.0 + jnp.exp(-v))


def _softplus(v):
    return jnp.maximum(v, 0.0) + jnp.log1p(jnp.exp(-jnp.abs(v)))


def _make_kernel(B, DM, DI, N, R, L, nch, t8):
    def body(x_ref, xh_ref, wiu_ref, wiz_ref, biu_ref, biz_ref, cw_ref,
             cb_ref, wdtr_ref, wb_ref, wc_ref, wdt_ref, dtb_ref, at_ref,
             d_ref, wo_ref, bo_ref, out_ref,
             g_s, uc_s, delta_s, bc_s, cc_s, y_s, h_s, ucar):
        j = pl.program_id(1)
        xc = x_ref[0]
        xb = xc.astype(jnp.bfloat16)

        u_raw = jnp.dot(xb, wiu_ref[...],
                        preferred_element_type=jnp.float32) + biu_ref[...]
        zv = jnp.dot(xb, wiz_ref[...],
                     preferred_element_type=jnp.float32) + biz_ref[...]
        g_s[...] = zv * _sigmoid(zv)

        prev = jnp.where(j == 0, 0.0, ucar[...])
        nxt = jnp.dot(xh_ref[0, 0:1, :].astype(jnp.bfloat16), wiu_ref[...],
                      preferred_element_type=jnp.float32) + biu_ref[...]
        nxt = jnp.where(j == nch - 1, 0.0, nxt)
        ucar[...] = u_raw[T - 1:T, :]
        u_dn = jnp.concatenate([prev, u_raw[:T - 1, :]], axis=0)
        u_up = jnp.concatenate([u_raw[1:, :], nxt], axis=0)
        ucv = (u_dn * cw_ref[0:1, :] + u_raw * cw_ref[1:2, :]
               + u_up * cw_ref[2:3, :] + cb_ref[...])
        ucv = ucv * _sigmoid(ucv)
        uc_s[...] = ucv.reshape(t8, SUB, DI)
        ucb = ucv.astype(jnp.bfloat16)

        dn = (((1,), (1,)), ((), ()))
        dtr = lax.dot_general(ucb, wdtr_ref[...], dn,
                              preferred_element_type=jnp.float32)
        bc = lax.dot_general(ucb, wb_ref[...], dn,
                             preferred_element_type=jnp.float32)
        cc = lax.dot_general(ucb, wc_ref[...], dn,
                             preferred_element_type=jnp.float32)
        bc_s[...] = bc.reshape(t8, SUB, N)
        cc_s[...] = cc.reshape(t8, SUB, N)
        dtpre = jnp.dot(dtr.astype(jnp.bfloat16), wdt_ref[...],
                        preferred_element_type=jnp.float32) + 2.0 * dtb_ref[...]
        delta_s[...] = _softplus(dtpre).reshape(t8, SUB, DI)

        aneg = -jnp.exp(at_ref[...])

        @pl.when(j == 0)
        def _():
            h_s[...] = jnp.zeros_like(h_s)

        def block(tb, h):
            d8 = delta_s[tb]
            u8 = uc_s[tb]
            b8 = bc_s[tb]
            c8 = cc_s[tb]
            du8 = d8 * u8
            da8 = jnp.exp(d8[:, None, :] * aneg[None, :, :])
            db8 = du8[:, None, :] * b8[:, :, None]
            c83 = c8[:, :, None]
            rows = []
            for r in range(SUB):
                h = da8[r] * h + db8[r]
                rows.append(jnp.sum(h * c83[r], axis=0, keepdims=True))
            y_s[tb] = jnp.concatenate(rows, axis=0)
            return h

        h = lax.fori_loop(0, t8, block, h_s[...])
        h_s[...] = h

        y = y_s[...].reshape(T, DI)
        yg = (y + d_ref[...] * uc_s[...].reshape(T, DI)) * g_s[...]
        out_ref[0] = (jnp.dot(yg.astype(jnp.bfloat16), wo_ref[...],
                              preferred_element_type=jnp.float32)
                      + bo_ref[...] + xc)
    return body


def kernel(x, in_proj_w, in_proj_b, conv_w, conv_b, x_proj_w, dt_proj_w,
           dt_proj_b, A_log, D, out_proj_w, out_proj_b):
    B, DM, L = x.shape
    DI = in_proj_w.shape[0] // 2
    R = dt_proj_w.shape[1]
    N = (x_proj_w.shape[0] - R) // 2
    nch = L // T
    t8 = T // SUB

    x_t = jnp.transpose(x, (0, 2, 1))
    bf = jnp.bfloat16
    wiu = jnp.transpose(in_proj_w[:DI], (1, 0)).astype(bf)
    wiz = jnp.transpose(in_proj_w[DI:], (1, 0)).astype(bf)
    biu = in_proj_b[:DI][None, :]
    biz = in_proj_b[DI:][None, :]
    cw = jnp.transpose(conv_w[:, 0, :], (1, 0))
    cb = conv_b[None, :]
    wdtr = x_proj_w[:R].astype(bf)
    wb = x_proj_w[R:R + N].astype(bf)
    wc = x_proj_w[R + N:].astype(bf)
    wdt = jnp.transpose(dt_proj_w, (1, 0)).astype(bf)
    dtb = dt_proj_b[None, :]
    at = jnp.transpose(A_log, (1, 0))
    drow = D[None, :]
    wo = jnp.transpose(out_proj_w, (1, 0)).astype(bf)
    bo = out_proj_b[None, :]

    full = lambda s: pl.BlockSpec(s, lambda b, j: tuple(0 for _ in s))
    out_t = pl.pallas_call(
        _make_kernel(B, DM, DI, N, R, L, nch, t8),
        out_shape=jax.ShapeDtypeStruct((B, L, DM), jnp.float32),
        grid=(B, nch),
        in_specs=[
            pl.BlockSpec((1, T, DM), lambda b, j: (b, j, 0)),
            pl.BlockSpec((1, 8, DM),
                         lambda b, j: (b, jnp.minimum((j + 1) * (T // 8),
                                                      L // 8 - 1), 0)),
            full((DM, DI)), full((DM, DI)), full((1, DI)), full((1, DI)),
            full((3, DI)), full((1, DI)), full((R, DI)), full((N, DI)),
            full((N, DI)), full((R, DI)), full((1, DI)), full((N, DI)),
            full((1, DI)), full((DI, DM)), full((1, DM)),
        ],
        out_specs=pl.BlockSpec((1, T, DM), lambda b, j: (b, j, 0)),
        scratch_shapes=[
            pltpu.VMEM((T, DI), jnp.float32),
            pltpu.VMEM((t8, SUB, DI), jnp.float32),
            pltpu.VMEM((t8, SUB, DI), jnp.float32),
            pltpu.VMEM((t8, SUB, N), jnp.float32),
            pltpu.VMEM((t8, SUB, N), jnp.float32),
            pltpu.VMEM((t8, SUB, DI), jnp.float32),
            pltpu.VMEM((N, DI), jnp.float32),
            pltpu.VMEM((1, DI), jnp.float32),
        ],
        compiler_params=pltpu.CompilerParams(
            dimension_semantics=("parallel", "arbitrary"),
            vmem_limit_bytes=64 * 1024 * 1024,
        ),
        name="vsss_block1d",
    )(x_t, x_t, wiu, wiz, biu, biz, cw, cb, wdtr, wb, wc, wdt, dtb, at,
      drow, wo, bo)
    return jnp.transpose(out_t, (0, 2, 1))


# speedup vs baseline: 1.1503x; 1.0209x over previous
import jax
import jax.numpy as jnp
from jax import lax
from jax.experimental import pallas as pl
from jax.experimental.pallas import tpu as pltpu

T = 256
SUB = 8
G = 2


def _sigmoid(v):
    return 1.0 / (1.0 + jnp.exp(-v))


def _softplus(v):
    return jnp.maximum(v, 0.0) + jnp.log1p(jnp.exp(-jnp.abs(v)))


def _make_kernel(B, DM, DI, N, R, L, nch, t8):
    def body(x_ref, xh_ref, wiu_ref, wiz_ref, biu_ref, biz_ref, cw_ref,
             cb_ref, wdtr_ref, wb_ref, wc_ref, wdt_ref, dtb_ref, at_ref,
             d_ref, wo_ref, bo_ref, out_ref,
             g_s, uc_s, delta_s, bc_s, cc_s, y_s, h_s, ucar):
        j = pl.program_id(1)
        xc = x_ref[...].reshape(G * T, DM)
        xb = xc.astype(jnp.bfloat16)

        u_raw = jnp.dot(xb, wiu_ref[...],
                        preferred_element_type=jnp.float32) + biu_ref[...]
        zv = jnp.dot(xb, wiz_ref[...],
                     preferred_element_type=jnp.float32) + biz_ref[...]
        g_s[...] = zv * _sigmoid(zv)

        nxt = jnp.dot(xh_ref[:, 0, :].astype(jnp.bfloat16), wiu_ref[...],
                      preferred_element_type=jnp.float32) + biu_ref[...]
        nxt = jnp.where(j == nch - 1, 0.0, nxt)
        prev = jnp.where(j == 0, 0.0, ucar[...])
        parts_dn, parts_up = [], []
        for g in range(G):
            ug = u_raw[g * T:(g + 1) * T]
            parts_dn.append(prev[g:g + 1])
            parts_dn.append(ug[:T - 1])
            parts_up.append(ug[1:])
            parts_up.append(nxt[g:g + 1])
        ucar[...] = jnp.concatenate(
            [u_raw[(g + 1) * T - 1:(g + 1) * T] for g in range(G)], axis=0)
        u_dn = jnp.concatenate(parts_dn, axis=0)
        u_up = jnp.concatenate(parts_up, axis=0)
        ucv = (u_dn * cw_ref[0:1, :] + u_raw * cw_ref[1:2, :]
               + u_up * cw_ref[2:3, :] + cb_ref[...])
        ucv = ucv * _sigmoid(ucv)
        uc_s[...] = ucv.reshape(G, t8, SUB, DI)
        ucb = ucv.astype(jnp.bfloat16)

        dn = (((1,), (1,)), ((), ()))
        dtr = lax.dot_general(ucb, wdtr_ref[...], dn,
                              preferred_element_type=jnp.float32)
        bc = lax.dot_general(ucb, wb_ref[...], dn,
                             preferred_element_type=jnp.float32)
        cc = lax.dot_general(ucb, wc_ref[...], dn,
                             preferred_element_type=jnp.float32)
        bc_s[...] = bc.reshape(G, t8, SUB, N)
        cc_s[...] = cc.reshape(G, t8, SUB, N)
        dtpre = jnp.dot(dtr.astype(jnp.bfloat16), wdt_ref[...],
                        preferred_element_type=jnp.float32) + 2.0 * dtb_ref[...]
        delta_s[...] = _softplus(dtpre).reshape(G, t8, SUB, DI)

        aneg = -jnp.exp(at_ref[...])

        @pl.when(j == 0)
        def _():
            h_s[...] = jnp.zeros_like(h_s)

        def block(tb, carry):
            hs = list(carry)
            for g in range(G):
                d8 = delta_s[g, tb]
                u8 = uc_s[g, tb]
                b8 = bc_s[g, tb]
                c8 = cc_s[g, tb]
                du8 = d8 * u8
                da8 = jnp.exp(d8[:, None, :] * aneg[None, :, :])
                db8 = du8[:, None, :] * b8[:, :, None]
                c83 = c8[:, :, None]
                h = hs[g]
                rows = []
                for r in range(SUB):
                    h = da8[r] * h + db8[r]
                    rows.append(jnp.sum(h * c83[r], axis=0, keepdims=True))
                y_s[g, tb] = jnp.concatenate(rows, axis=0)
                hs[g] = h
            return tuple(hs)

        h0 = tuple(h_s[g] for g in range(G))
        hf = lax.fori_loop(0, t8, block, h0)
        for g in range(G):
            h_s[g] = hf[g]

        y = y_s[...].reshape(G * T, DI)
        yg = (y + d_ref[...] * uc_s[...].reshape(G * T, DI)) * g_s[...]
        out_ref[...] = (jnp.dot(yg.astype(jnp.bfloat16), wo_ref[...],
                                preferred_element_type=jnp.float32)
                        + bo_ref[...] + xc).reshape(G, T, DM)
    return body


def kernel(x, in_proj_w, in_proj_b, conv_w, conv_b, x_proj_w, dt_proj_w,
           dt_proj_b, A_log, D, out_proj_w, out_proj_b):
    B, DM, L = x.shape
    DI = in_proj_w.shape[0] // 2
    R = dt_proj_w.shape[1]
    N = (x_proj_w.shape[0] - R) // 2
    nch = L // T
    t8 = T // SUB

    x_t = jnp.transpose(x, (0, 2, 1))
    bf = jnp.bfloat16
    wiu = jnp.transpose(in_proj_w[:DI], (1, 0)).astype(bf)
    wiz = jnp.transpose(in_proj_w[DI:], (1, 0)).astype(bf)
    biu = in_proj_b[:DI][None, :]
    biz = in_proj_b[DI:][None, :]
    cw = jnp.transpose(conv_w[:, 0, :], (1, 0))
    cb = conv_b[None, :]
    wdtr = x_proj_w[:R].astype(bf)
    wb = x_proj_w[R:R + N].astype(bf)
    wc = x_proj_w[R + N:].astype(bf)
    wdt = jnp.transpose(dt_proj_w, (1, 0)).astype(bf)
    dtb = dt_proj_b[None, :]
    at = jnp.transpose(A_log, (1, 0))
    drow = D[None, :]
    wo = jnp.transpose(out_proj_w, (1, 0)).astype(bf)
    bo = out_proj_b[None, :]

    full = lambda s: pl.BlockSpec(s, lambda b, j: tuple(0 for _ in s))
    out_t = pl.pallas_call(
        _make_kernel(B, DM, DI, N, R, L, nch, t8),
        out_shape=jax.ShapeDtypeStruct((B, L, DM), jnp.float32),
        grid=(B // G, nch),
        in_specs=[
            pl.BlockSpec((G, T, DM), lambda b, j: (b, j, 0)),
            pl.BlockSpec((G, 8, DM),
                         lambda b, j: (b, jnp.minimum((j + 1) * (T // 8),
                                                      L // 8 - 1), 0)),
            full((DM, DI)), full((DM, DI)), full((1, DI)), full((1, DI)),
            full((3, DI)), full((1, DI)), full((R, DI)), full((N, DI)),
            full((N, DI)), full((R, DI)), full((1, DI)), full((N, DI)),
            full((1, DI)), full((DI, DM)), full((1, DM)),
        ],
        out_specs=pl.BlockSpec((G, T, DM), lambda b, j: (b, j, 0)),
        scratch_shapes=[
            pltpu.VMEM((G * T, DI), jnp.float32),
            pltpu.VMEM((G, t8, SUB, DI), jnp.float32),
            pltpu.VMEM((G, t8, SUB, DI), jnp.float32),
            pltpu.VMEM((G, t8, SUB, N), jnp.float32),
            pltpu.VMEM((G, t8, SUB, N), jnp.float32),
            pltpu.VMEM((G, t8, SUB, DI), jnp.float32),
            pltpu.VMEM((G, N, DI), jnp.float32),
            pltpu.VMEM((G, DI), jnp.float32),
        ],
        compiler_params=pltpu.CompilerParams(
            dimension_semantics=("parallel", "arbitrary"),
            vmem_limit_bytes=64 * 1024 * 1024,
        ),
        name="vsss_block1d",
    )(x_t, x_t, wiu, wiz, biu, biz, cw, cb, wdtr, wb, wc, wdt, dtb, at,
      drow, wo, bo)
    return jnp.transpose(out_t, (0, 2, 1))


# speedup vs baseline: 1.1798x; 1.0256x over previous
import jax
import jax.numpy as jnp
from jax import lax
from jax.experimental import pallas as pl
from jax.experimental.pallas import tpu as pltpu

T = 256
SUB = 8
G = 2


def _sigmoid(v):
    return 1.0 / (1.0 + jnp.exp(-v))


def _softplus(v):
    return jnp.maximum(v, 0.0) + jnp.log1p(jnp.exp(-jnp.abs(v)))


def _make_kernel(B, DM, DI, N, R, L, nch, t8):
    def body(x_ref, xh_ref, wiu_ref, wiz_ref, biu_ref, biz_ref, cw_ref,
             cb_ref, wdtr_ref, wb_ref, wc_ref, wdt_ref, dtb_ref, at_ref,
             d_ref, wo_ref, bo_ref, out_ref,
             g_s, uc_s, delta_s, bc_s, cc_s, y_s, h_s, ucar):
        j = pl.program_id(1)
        xc = x_ref[...].reshape(G * T, DM)
        xb = xc.astype(jnp.bfloat16)

        u_raw = jnp.dot(xb, wiu_ref[...],
                        preferred_element_type=jnp.float32) + biu_ref[...]
        zv = jnp.dot(xb, wiz_ref[...],
                     preferred_element_type=jnp.float32) + biz_ref[...]
        g_s[...] = zv * _sigmoid(zv)

        nxt = jnp.dot(xh_ref[:, 0, :].astype(jnp.bfloat16), wiu_ref[...],
                      preferred_element_type=jnp.float32) + biu_ref[...]
        nxt = jnp.where(j == nch - 1, 0.0, nxt)
        prev = jnp.where(j == 0, 0.0, ucar[...])
        parts_dn, parts_up = [], []
        for g in range(G):
            ug = u_raw[g * T:(g + 1) * T]
            parts_dn.append(prev[g:g + 1])
            parts_dn.append(ug[:T - 1])
            parts_up.append(ug[1:])
            parts_up.append(nxt[g:g + 1])
        ucar[...] = jnp.concatenate(
            [u_raw[(g + 1) * T - 1:(g + 1) * T] for g in range(G)], axis=0)
        u_dn = jnp.concatenate(parts_dn, axis=0)
        u_up = jnp.concatenate(parts_up, axis=0)
        ucv = (u_dn * cw_ref[0:1, :] + u_raw * cw_ref[1:2, :]
               + u_up * cw_ref[2:3, :] + cb_ref[...])
        ucv = ucv * _sigmoid(ucv)
        uc_s[...] = ucv.reshape(G, t8, SUB, DI)
        ucb = ucv.astype(jnp.bfloat16)

        dn = (((1,), (1,)), ((), ()))
        dtr = lax.dot_general(ucb, wdtr_ref[...], dn,
                              preferred_element_type=jnp.float32)
        bc = lax.dot_general(ucb, wb_ref[...], dn,
                             preferred_element_type=jnp.float32)
        cc = lax.dot_general(ucb, wc_ref[...], dn,
                             preferred_element_type=jnp.float32)
        bc_s[...] = bc.reshape(G, t8, SUB, N)
        cc_s[...] = cc.reshape(G, t8, SUB, N)
        dtpre = jnp.dot(dtr.astype(jnp.bfloat16), wdt_ref[...],
                        preferred_element_type=jnp.float32) + 2.0 * dtb_ref[...]
        delta_s[...] = _softplus(dtpre).reshape(G, t8, SUB, DI)

        aneg = -jnp.exp(at_ref[...])

        @pl.when(j == 0)
        def _():
            h_s[...] = jnp.zeros_like(h_s)

        def block(tb, carry):
            hs = list(carry)
            for g in range(G):
                d8 = delta_s[g, tb]
                u8 = uc_s[g, tb]
                b8 = bc_s[g, tb]
                c8 = cc_s[g, tb]
                du8 = d8 * u8
                c83 = c8[:, :, None]
                h = hs[g]
                rows = []
                for rr in range(0, SUB, 4):
                    dg = d8[rr:rr + 4]
                    da4 = jnp.exp(dg[:, None, :] * aneg[None, :, :])
                    db4 = (du8[rr:rr + 4, None, :]
                           * b8[rr:rr + 4, :, None])
                    for r in range(4):
                        h = da4[r] * h + db4[r]
                        rows.append(jnp.sum(h * c83[rr + r], axis=0,
                                            keepdims=True))
                y_s[g, tb] = jnp.concatenate(rows, axis=0)
                hs[g] = h
            return tuple(hs)

        h0 = tuple(h_s[g] for g in range(G))
        hf = lax.fori_loop(0, t8, block, h0)
        for g in range(G):
            h_s[g] = hf[g]

        y = y_s[...].reshape(G * T, DI)
        yg = (y + d_ref[...] * uc_s[...].reshape(G * T, DI)) * g_s[...]
        out_ref[...] = (jnp.dot(yg.astype(jnp.bfloat16), wo_ref[...],
                                preferred_element_type=jnp.float32)
                        + bo_ref[...] + xc).reshape(G, T, DM)
    return body


def kernel(x, in_proj_w, in_proj_b, conv_w, conv_b, x_proj_w, dt_proj_w,
           dt_proj_b, A_log, D, out_proj_w, out_proj_b):
    B, DM, L = x.shape
    DI = in_proj_w.shape[0] // 2
    R = dt_proj_w.shape[1]
    N = (x_proj_w.shape[0] - R) // 2
    nch = L // T
    t8 = T // SUB

    x_t = jnp.transpose(x, (0, 2, 1))
    bf = jnp.bfloat16
    wiu = jnp.transpose(in_proj_w[:DI], (1, 0)).astype(bf)
    wiz = jnp.transpose(in_proj_w[DI:], (1, 0)).astype(bf)
    biu = in_proj_b[:DI][None, :]
    biz = in_proj_b[DI:][None, :]
    cw = jnp.transpose(conv_w[:, 0, :], (1, 0))
    cb = conv_b[None, :]
    wdtr = x_proj_w[:R].astype(bf)
    wb = x_proj_w[R:R + N].astype(bf)
    wc = x_proj_w[R + N:].astype(bf)
    wdt = jnp.transpose(dt_proj_w, (1, 0)).astype(bf)
    dtb = dt_proj_b[None, :]
    at = jnp.transpose(A_log, (1, 0))
    drow = D[None, :]
    wo = jnp.transpose(out_proj_w, (1, 0)).astype(bf)
    bo = out_proj_b[None, :]

    full = lambda s: pl.BlockSpec(s, lambda b, j: tuple(0 for _ in s))
    out_t = pl.pallas_call(
        _make_kernel(B, DM, DI, N, R, L, nch, t8),
        out_shape=jax.ShapeDtypeStruct((B, L, DM), jnp.float32),
        grid=(B // G, nch),
        in_specs=[
            pl.BlockSpec((G, T, DM), lambda b, j: (b, j, 0)),
            pl.BlockSpec((G, 8, DM),
                         lambda b, j: (b, jnp.minimum((j + 1) * (T // 8),
                                                      L // 8 - 1), 0)),
            full((DM, DI)), full((DM, DI)), full((1, DI)), full((1, DI)),
            full((3, DI)), full((1, DI)), full((R, DI)), full((N, DI)),
            full((N, DI)), full((R, DI)), full((1, DI)), full((N, DI)),
            full((1, DI)), full((DI, DM)), full((1, DM)),
        ],
        out_specs=pl.BlockSpec((G, T, DM), lambda b, j: (b, j, 0)),
        scratch_shapes=[
            pltpu.VMEM((G * T, DI), jnp.float32),
            pltpu.VMEM((G, t8, SUB, DI), jnp.float32),
            pltpu.VMEM((G, t8, SUB, DI), jnp.float32),
            pltpu.VMEM((G, t8, SUB, N), jnp.float32),
            pltpu.VMEM((G, t8, SUB, N), jnp.float32),
            pltpu.VMEM((G, t8, SUB, DI), jnp.float32),
            pltpu.VMEM((G, N, DI), jnp.float32),
            pltpu.VMEM((G, DI), jnp.float32),
        ],
        compiler_params=pltpu.CompilerParams(
            dimension_semantics=("parallel", "arbitrary"),
            vmem_limit_bytes=64 * 1024 * 1024,
        ),
        name="vsss_block1d",
    )(x_t, x_t, wiu, wiz, biu, biz, cw, cb, wdtr, wb, wc, wdt, dtb, at,
      drow, wo, bo)
    return jnp.transpose(out_t, (0, 2, 1))


# speedup vs baseline: 1.1826x; 1.0024x over previous
import jax
import jax.numpy as jnp
from jax import lax
from jax.experimental import pallas as pl
from jax.experimental.pallas import tpu as pltpu

T = 256
SUB = 8
G = 2


def _sigmoid(v):
    return 1.0 / (1.0 + jnp.exp(-v))


def _softplus(v):
    return jnp.maximum(v, 0.0) + jnp.log1p(jnp.exp(-jnp.abs(v)))


def _make_kernel(B, DM, DI, N, R, L, nch, t8):
    def body(x_ref, xh_ref, wiu_ref, wiz_ref, biu_ref, biz_ref, cw_ref,
             cb_ref, wdtr_ref, wb_ref, wc_ref, wdt_ref, dtb_ref, at_ref,
             d_ref, wo_ref, bo_ref, out_ref,
             g_s, uc_s, delta_s, bc_s, cc_s, y_s, h_s, ucar):
        j = pl.program_id(1)
        xc = x_ref[...].reshape(G * T, DM)
        xb = xc.astype(jnp.bfloat16)

        u_raw = jnp.dot(xb, wiu_ref[...],
                        preferred_element_type=jnp.float32) + biu_ref[...]
        zv = jnp.dot(xb, wiz_ref[...],
                     preferred_element_type=jnp.float32) + biz_ref[...]
        g_s[...] = zv * _sigmoid(zv)

        nxt = jnp.dot(xh_ref[:, 0, :].astype(jnp.bfloat16), wiu_ref[...],
                      preferred_element_type=jnp.float32) + biu_ref[...]
        nxt = jnp.where(j == nch - 1, 0.0, nxt)
        prev = jnp.where(j == 0, 0.0, ucar[...])
        parts_dn, parts_up = [], []
        for g in range(G):
            ug = u_raw[g * T:(g + 1) * T]
            parts_dn.append(prev[g:g + 1])
            parts_dn.append(ug[:T - 1])
            parts_up.append(ug[1:])
            parts_up.append(nxt[g:g + 1])
        ucar[...] = jnp.concatenate(
            [u_raw[(g + 1) * T - 1:(g + 1) * T] for g in range(G)], axis=0)
        u_dn = jnp.concatenate(parts_dn, axis=0)
        u_up = jnp.concatenate(parts_up, axis=0)
        ucv = (u_dn * cw_ref[0:1, :] + u_raw * cw_ref[1:2, :]
               + u_up * cw_ref[2:3, :] + cb_ref[...])
        ucv = ucv * _sigmoid(ucv)
        uc_s[...] = ucv.reshape(G, t8, SUB, DI)
        ucb = ucv.astype(jnp.bfloat16)

        dn = (((1,), (1,)), ((), ()))
        dtr = lax.dot_general(ucb, wdtr_ref[...], dn,
                              preferred_element_type=jnp.float32)
        bc = lax.dot_general(ucb, wb_ref[...], dn,
                             preferred_element_type=jnp.float32)
        cc = lax.dot_general(ucb, wc_ref[...], dn,
                             preferred_element_type=jnp.float32)
        bc_s[...] = bc.reshape(G, t8, SUB, N)
        cc_s[...] = cc.reshape(G, t8, SUB, N)
        dtpre = jnp.dot(dtr.astype(jnp.bfloat16), wdt_ref[...],
                        preferred_element_type=jnp.float32) + 2.0 * dtb_ref[...]
        delta_s[...] = _softplus(dtpre).reshape(G, t8, SUB, DI)

        aneg = -jnp.exp(at_ref[...])

        @pl.when(j == 0)
        def _():
            h_s[...] = jnp.zeros_like(h_s)

        def block(tb, carry):
            hs = list(carry)
            for g in range(G):
                d8 = delta_s[g, tb]
                u8 = uc_s[g, tb]
                b8 = bc_s[g, tb]
                c8 = cc_s[g, tb]
                du8 = d8 * u8
                c83 = c8[:, :, None]
                h = hs[g]
                rows = []
                for rr in range(0, SUB, 2):
                    dg = d8[rr:rr + 2]
                    da4 = jnp.exp(dg[:, None, :] * aneg[None, :, :])
                    db4 = (du8[rr:rr + 2, None, :]
                           * b8[rr:rr + 2, :, None])
                    for r in range(2):
                        h = da4[r] * h + db4[r]
                        rows.append(jnp.sum(h * c83[rr + r], axis=0,
                                            keepdims=True))
                y_s[g, tb] = jnp.concatenate(rows, axis=0)
                hs[g] = h
            return tuple(hs)

        h0 = tuple(h_s[g] for g in range(G))
        hf = lax.fori_loop(0, t8, block, h0)
        for g in range(G):
            h_s[g] = hf[g]

        y = y_s[...].reshape(G * T, DI)
        yg = (y + d_ref[...] * uc_s[...].reshape(G * T, DI)) * g_s[...]
        out_ref[...] = (jnp.dot(yg.astype(jnp.bfloat16), wo_ref[...],
                                preferred_element_type=jnp.float32)
                        + bo_ref[...] + xc).reshape(G, T, DM)
    return body


def kernel(x, in_proj_w, in_proj_b, conv_w, conv_b, x_proj_w, dt_proj_w,
           dt_proj_b, A_log, D, out_proj_w, out_proj_b):
    B, DM, L = x.shape
    DI = in_proj_w.shape[0] // 2
    R = dt_proj_w.shape[1]
    N = (x_proj_w.shape[0] - R) // 2
    nch = L // T
    t8 = T // SUB

    x_t = jnp.transpose(x, (0, 2, 1))
    bf = jnp.bfloat16
    wiu = jnp.transpose(in_proj_w[:DI], (1, 0)).astype(bf)
    wiz = jnp.transpose(in_proj_w[DI:], (1, 0)).astype(bf)
    biu = in_proj_b[:DI][None, :]
    biz = in_proj_b[DI:][None, :]
    cw = jnp.transpose(conv_w[:, 0, :], (1, 0))
    cb = conv_b[None, :]
    wdtr = x_proj_w[:R].astype(bf)
    wb = x_proj_w[R:R + N].astype(bf)
    wc = x_proj_w[R + N:].astype(bf)
    wdt = jnp.transpose(dt_proj_w, (1, 0)).astype(bf)
    dtb = dt_proj_b[None, :]
    at = jnp.transpose(A_log, (1, 0))
    drow = D[None, :]
    wo = jnp.transpose(out_proj_w, (1, 0)).astype(bf)
    bo = out_proj_b[None, :]

    full = lambda s: pl.BlockSpec(s, lambda b, j: tuple(0 for _ in s))
    out_t = pl.pallas_call(
        _make_kernel(B, DM, DI, N, R, L, nch, t8),
        out_shape=jax.ShapeDtypeStruct((B, L, DM), jnp.float32),
        grid=(B // G, nch),
        in_specs=[
            pl.BlockSpec((G, T, DM), lambda b, j: (b, j, 0)),
            pl.BlockSpec((G, 8, DM),
                         lambda b, j: (b, jnp.minimum((j + 1) * (T // 8),
                                                      L // 8 - 1), 0)),
            full((DM, DI)), full((DM, DI)), full((1, DI)), full((1, DI)),
            full((3, DI)), full((1, DI)), full((R, DI)), full((N, DI)),
            full((N, DI)), full((R, DI)), full((1, DI)), full((N, DI)),
            full((1, DI)), full((DI, DM)), full((1, DM)),
        ],
        out_specs=pl.BlockSpec((G, T, DM), lambda b, j: (b, j, 0)),
        scratch_shapes=[
            pltpu.VMEM((G * T, DI), jnp.float32),
            pltpu.VMEM((G, t8, SUB, DI), jnp.float32),
            pltpu.VMEM((G, t8, SUB, DI), jnp.float32),
            pltpu.VMEM((G, t8, SUB, N), jnp.float32),
            pltpu.VMEM((G, t8, SUB, N), jnp.float32),
            pltpu.VMEM((G, t8, SUB, DI), jnp.float32),
            pltpu.VMEM((G, N, DI), jnp.float32),
            pltpu.VMEM((G, DI), jnp.float32),
        ],
        compiler_params=pltpu.CompilerParams(
            dimension_semantics=("parallel", "arbitrary"),
            vmem_limit_bytes=64 * 1024 * 1024,
        ),
        name="vsss_block1d",
    )(x_t, x_t, wiu, wiz, biu, biz, cw, cb, wdtr, wb, wc, wdt, dtb, at,
      drow, wo, bo)
    return jnp.transpose(out_t, (0, 2, 1))


# speedup vs baseline: 1.2118x; 1.0248x over previous
import jax
import jax.numpy as jnp
from jax import lax
from jax.experimental import pallas as pl
from jax.experimental.pallas import tpu as pltpu

T = 512
SUB = 8
G = 2


def _sigmoid(v):
    return 1.0 / (1.0 + jnp.exp(-v))


def _softplus(v):
    return jnp.maximum(v, 0.0) + jnp.log1p(jnp.exp(-jnp.abs(v)))


def _make_kernel(B, DM, DI, N, R, L, nch, t8):
    def body(x_ref, xh_ref, wiu_ref, wiz_ref, biu_ref, biz_ref, cw_ref,
             cb_ref, wdtr_ref, wb_ref, wc_ref, wdt_ref, dtb_ref, at_ref,
             d_ref, wo_ref, bo_ref, out_ref,
             g_s, uc_s, delta_s, bc_s, cc_s, y_s, h_s, ucar):
        j = pl.program_id(1)
        xc = x_ref[...].reshape(G * T, DM)
        xb = xc.astype(jnp.bfloat16)

        u_raw = jnp.dot(xb, wiu_ref[...],
                        preferred_element_type=jnp.float32) + biu_ref[...]
        zv = jnp.dot(xb, wiz_ref[...],
                     preferred_element_type=jnp.float32) + biz_ref[...]
        g_s[...] = zv * _sigmoid(zv)

        nxt = jnp.dot(xh_ref[:, 0, :].astype(jnp.bfloat16), wiu_ref[...],
                      preferred_element_type=jnp.float32) + biu_ref[...]
        nxt = jnp.where(j == nch - 1, 0.0, nxt)
        prev = jnp.where(j == 0, 0.0, ucar[...])
        parts_dn, parts_up = [], []
        for g in range(G):
            ug = u_raw[g * T:(g + 1) * T]
            parts_dn.append(prev[g:g + 1])
            parts_dn.append(ug[:T - 1])
            parts_up.append(ug[1:])
            parts_up.append(nxt[g:g + 1])
        ucar[...] = jnp.concatenate(
            [u_raw[(g + 1) * T - 1:(g + 1) * T] for g in range(G)], axis=0)
        u_dn = jnp.concatenate(parts_dn, axis=0)
        u_up = jnp.concatenate(parts_up, axis=0)
        ucv = (u_dn * cw_ref[0:1, :] + u_raw * cw_ref[1:2, :]
               + u_up * cw_ref[2:3, :] + cb_ref[...])
        ucv = ucv * _sigmoid(ucv)
        uc_s[...] = ucv.reshape(G, t8, SUB, DI)
        ucb = ucv.astype(jnp.bfloat16)

        dn = (((1,), (1,)), ((), ()))
        dtr = lax.dot_general(ucb, wdtr_ref[...], dn,
                              preferred_element_type=jnp.float32)
        bc = lax.dot_general(ucb, wb_ref[...], dn,
                             preferred_element_type=jnp.float32)
        cc = lax.dot_general(ucb, wc_ref[...], dn,
                             preferred_element_type=jnp.float32)
        bc_s[...] = bc.reshape(G, t8, SUB, N)
        cc_s[...] = cc.reshape(G, t8, SUB, N)
        dtpre = jnp.dot(dtr.astype(jnp.bfloat16), wdt_ref[...],
                        preferred_element_type=jnp.float32) + 2.0 * dtb_ref[...]
        delta_s[...] = _softplus(dtpre).reshape(G, t8, SUB, DI)

        aneg = -jnp.exp(at_ref[...])

        @pl.when(j == 0)
        def _():
            h_s[...] = jnp.zeros_like(h_s)

        def block(tb, carry):
            hs = list(carry)
            for g in range(G):
                d8 = delta_s[g, tb]
                u8 = uc_s[g, tb]
                b8 = bc_s[g, tb]
                c8 = cc_s[g, tb]
                du8 = d8 * u8
                c83 = c8[:, :, None]
                h = hs[g]
                rows = []
                for rr in range(0, SUB, 2):
                    dg = d8[rr:rr + 2]
                    da4 = jnp.exp(dg[:, None, :] * aneg[None, :, :])
                    db4 = (du8[rr:rr + 2, None, :]
                           * b8[rr:rr + 2, :, None])
                    for r in range(2):
                        h = da4[r] * h + db4[r]
                        rows.append(jnp.sum(h * c83[rr + r], axis=0,
                                            keepdims=True))
                y_s[g, tb] = jnp.concatenate(rows, axis=0)
                hs[g] = h
            return tuple(hs)

        h0 = tuple(h_s[g] for g in range(G))
        hf = lax.fori_loop(0, t8, block, h0)
        for g in range(G):
            h_s[g] = hf[g]

        y = y_s[...].reshape(G * T, DI)
        yg = (y + d_ref[...] * uc_s[...].reshape(G * T, DI)) * g_s[...]
        out_ref[...] = (jnp.dot(yg.astype(jnp.bfloat16), wo_ref[...],
                                preferred_element_type=jnp.float32)
                        + bo_ref[...] + xc).reshape(G, T, DM)
    return body


def kernel(x, in_proj_w, in_proj_b, conv_w, conv_b, x_proj_w, dt_proj_w,
           dt_proj_b, A_log, D, out_proj_w, out_proj_b):
    B, DM, L = x.shape
    DI = in_proj_w.shape[0] // 2
    R = dt_proj_w.shape[1]
    N = (x_proj_w.shape[0] - R) // 2
    nch = L // T
    t8 = T // SUB

    x_t = jnp.transpose(x, (0, 2, 1))
    bf = jnp.bfloat16
    wiu = jnp.transpose(in_proj_w[:DI], (1, 0)).astype(bf)
    wiz = jnp.transpose(in_proj_w[DI:], (1, 0)).astype(bf)
    biu = in_proj_b[:DI][None, :]
    biz = in_proj_b[DI:][None, :]
    cw = jnp.transpose(conv_w[:, 0, :], (1, 0))
    cb = conv_b[None, :]
    wdtr = x_proj_w[:R].astype(bf)
    wb = x_proj_w[R:R + N].astype(bf)
    wc = x_proj_w[R + N:].astype(bf)
    wdt = jnp.transpose(dt_proj_w, (1, 0)).astype(bf)
    dtb = dt_proj_b[None, :]
    at = jnp.transpose(A_log, (1, 0))
    drow = D[None, :]
    wo = jnp.transpose(out_proj_w, (1, 0)).astype(bf)
    bo = out_proj_b[None, :]

    full = lambda s: pl.BlockSpec(s, lambda b, j: tuple(0 for _ in s))
    out_t = pl.pallas_call(
        _make_kernel(B, DM, DI, N, R, L, nch, t8),
        out_shape=jax.ShapeDtypeStruct((B, L, DM), jnp.float32),
        grid=(B // G, nch),
        in_specs=[
            pl.BlockSpec((G, T, DM), lambda b, j: (b, j, 0)),
            pl.BlockSpec((G, 8, DM),
                         lambda b, j: (b, jnp.minimum((j + 1) * (T // 8),
                                                      L // 8 - 1), 0)),
            full((DM, DI)), full((DM, DI)), full((1, DI)), full((1, DI)),
            full((3, DI)), full((1, DI)), full((R, DI)), full((N, DI)),
            full((N, DI)), full((R, DI)), full((1, DI)), full((N, DI)),
            full((1, DI)), full((DI, DM)), full((1, DM)),
        ],
        out_specs=pl.BlockSpec((G, T, DM), lambda b, j: (b, j, 0)),
        scratch_shapes=[
            pltpu.VMEM((G * T, DI), jnp.float32),
            pltpu.VMEM((G, t8, SUB, DI), jnp.float32),
            pltpu.VMEM((G, t8, SUB, DI), jnp.float32),
            pltpu.VMEM((G, t8, SUB, N), jnp.float32),
            pltpu.VMEM((G, t8, SUB, N), jnp.float32),
            pltpu.VMEM((G, t8, SUB, DI), jnp.float32),
            pltpu.VMEM((G, N, DI), jnp.float32),
            pltpu.VMEM((G, DI), jnp.float32),
        ],
        compiler_params=pltpu.CompilerParams(
            dimension_semantics=("parallel", "arbitrary"),
            vmem_limit_bytes=64 * 1024 * 1024,
        ),
        name="vsss_block1d",
    )(x_t, x_t, wiu, wiz, biu, biz, cw, cb, wdtr, wb, wc, wdt, dtb, at,
      drow, wo, bo)
    return jnp.transpose(out_t, (0, 2, 1))
